# baseline (speedup 1.0000x reference)
import functools

import jax
import jax.numpy as jnp
from jax import lax
from jax.experimental import pallas as pl
from jax.experimental.pallas import tpu as pltpu
from jax.experimental.pallas import tpu_sc as plsc

N = 100000
D = 512
G = 64
L = 16
NT = 32
ACC_W = D + L

B_TC = 2048
R_TC = 57344
NBLK_TC = R_TC // B_TC
SC_OFF = R_TC
N_SC = N - R_TC

ROWS_PER_CHUNK = 32
N_CHUNKS = N_SC // ROWS_PER_CHUNK
BASE_CHUNKS = N_CHUNKS // NT
EXTRA_CHUNKS = N_CHUNKS % NT
NBUF = 4
MAX_CHUNKS = -(-(BASE_CHUNKS + 1) // NBUF) * NBUF
MAX_ROWS = MAX_CHUNKS * ROWS_PER_CHUNK


def _seg_body(z_hbm, batch_hbm, out_hbm, batch_v, zbuf, acc, *sems):
    cid = lax.axis_index("c")
    sid = lax.axis_index("s")
    wid = sid * 2 + cid
    n_chunks = BASE_CHUNKS + jnp.where(wid < EXTRA_CHUNKS, 1, 0)
    chunk0 = wid * BASE_CHUNKS + jnp.minimum(wid, EXTRA_CHUNKS)
    row0 = SC_OFF + chunk0 * ROWS_PER_CHUNK

    zero16 = jnp.zeros((L,), jnp.float32)

    def _zero_row(i, carry):
        for j in range(ACC_W // L):
            acc[i, pl.ds(j * L, L)] = zero16
        return carry

    lax.fori_loop(0, G + 1, _zero_row, 0)

    bstart = jnp.minimum(row0, N - MAX_ROWS)
    boff = row0 - bstart
    pltpu.sync_copy(batch_hbm.at[pl.ds(bstart, MAX_ROWS)],
                    batch_v.at[pl.ds(0, MAX_ROWS)])

    def _dma(j, buf):
        src = jnp.where(j < n_chunks,
                        SC_OFF + (chunk0 + j) * ROWS_PER_CHUNK,
                        N - ROWS_PER_CHUNK)
        return pltpu.make_async_copy(
            z_hbm.at[pl.ds(src, ROWS_PER_CHUNK)], zbuf.at[buf], sems[buf])

    def _process(i, buf):
        valid = i < n_chunks
        loff = jnp.where(valid, boff + i * ROWS_PER_CHUNK, 0)
        seg_lo = batch_v[pl.ds(loff, L)]
        seg_hi = batch_v[pl.ds(loff + ROWS_PER_CHUNK - L, L)]
        s_first = jnp.where(valid, seg_lo[0], G)
        s_last = jnp.where(valid, seg_hi[L - 1], G)

        @pl.when(s_first == s_last)
        def _fast():
            def _cols(c8, carry):
                for cc in range(8):
                    cs = pl.ds((c8 * 8 + cc) * L, L)
                    vals = [zbuf[buf, r, cs] for r in range(ROWS_PER_CHUNK)]
                    while len(vals) > 1:
                        vals = [vals[k] + vals[k + 1]
                                for k in range(0, len(vals), 2)]
                    acc[s_first, cs] += vals[0]
                return carry

            lax.fori_loop(0, D // L // 8, _cols, 0)
            acc[s_first, pl.ds(D, L)] += jnp.full(
                (L,), float(ROWS_PER_CHUNK), jnp.float32)

        @pl.when(s_first != s_last)
        def _slow():
            def _row(r, carry):
                s_r = batch_v[pl.ds(loff + r, L)][0]

                def _cols(c8, c2):
                    for cc in range(8):
                        cs = pl.ds((c8 * 8 + cc) * L, L)
                        acc[s_r, cs] += zbuf[buf, r, cs]
                    return c2

                lax.fori_loop(0, D // L // 8, _cols, 0)
                acc[s_r, pl.ds(D, L)] += jnp.full((L,), 1.0, jnp.float32)
                return carry

            lax.fori_loop(0, ROWS_PER_CHUNK, _row, 0)

    for b in range(NBUF):
        _dma(b, b).start()

    def _quad(p, carry):
        i0 = NBUF * p
        for b in range(NBUF):
            _dma(i0 + b, b).wait()
            _process(i0 + b, b)

            @pl.when(i0 + b + NBUF < MAX_CHUNKS)
            def _():
                _dma(i0 + b + NBUF, b).start()

        return carry

    lax.fori_loop(0, MAX_CHUNKS // NBUF, _quad, 0)

    pltpu.sync_copy(acc.at[pl.ds(0, G)], out_hbm.at[wid])


_seg_kernel = functools.partial(
    pl.kernel,
    out_type=jax.ShapeDtypeStruct((NT, G, ACC_W), jnp.float32),
    mesh=plsc.VectorSubcoreMesh(core_axis_name="c", subcore_axis_name="s"),
    scratch_types=[
        pltpu.VMEM((MAX_ROWS + L,), jnp.int32),
        pltpu.VMEM((NBUF, ROWS_PER_CHUNK, D), jnp.float32),
        pltpu.VMEM((G + 1, ACC_W), jnp.float32),
    ] + [pltpu.SemaphoreType.DMA] * NBUF,
)(_seg_body)


def _tc_seg_body(b_ref, z_ref, sums_ref, cnt_ref):
    @pl.when(pl.program_id(0) == 0)
    def _():
        sums_ref[...] = jnp.zeros_like(sums_ref)
        cnt_ref[...] = jnp.zeros_like(cnt_ref)

    seg = b_ref[0, 0, :]
    gid = lax.broadcasted_iota(jnp.int32, (G, B_TC), 0)
    onehot = (jnp.broadcast_to(seg[None, :], (G, B_TC)) == gid
              ).astype(jnp.float32)
    sums_ref[...] += lax.dot_general(
        onehot.astype(jnp.bfloat16), z_ref[...].astype(jnp.bfloat16),
        (((1,), (0,)), ((), ())),
        preferred_element_type=jnp.float32)
    cnt128 = onehot[:, 0:128]
    for q in range(1, B_TC // 128):
        cnt128 = cnt128 + onehot[:, q * 128:(q + 1) * 128]
    cnt_ref[...] += cnt128


def _tc_seg(z, batch3d):
    return pl.pallas_call(
        _tc_seg_body,
        grid=(NBLK_TC,),
        in_specs=[
            pl.BlockSpec((1, 1, B_TC), lambda k: (k, 0, 0)),
            pl.BlockSpec((B_TC, D), lambda k: (k, 0)),
        ],
        out_specs=[
            pl.BlockSpec((G, D), lambda k: (0, 0)),
            pl.BlockSpec((G, 128), lambda k: (0, 0)),
        ],
        out_shape=[
            jax.ShapeDtypeStruct((G, D), jnp.float32),
            jax.ShapeDtypeStruct((G, 128), jnp.float32),
        ],
    )(batch3d, z)


def _mlp_body(p_ref, tcs_ref, tcc_ref, w1_ref, b1_ref, w2p_ref, b2p_ref,
              o_ref):
    total = jnp.sum(p_ref[...], axis=0)
    sums = total[:, :D] + tcs_ref[...]
    cnt = total[:, D:D + 1] + jnp.sum(tcc_ref[...], axis=1, keepdims=True)
    mean = sums / jnp.maximum(cnt, 1.0)
    h = lax.dot_general(mean, w1_ref[...], (((1,), (1,)), ((), ())),
                        preferred_element_type=jnp.float32) + b1_ref[...]
    h = jnp.maximum(h, 0.0)
    y = lax.dot_general(h, w2p_ref[...], (((1,), (0,)), ((), ())),
                        preferred_element_type=jnp.float32) + b2p_ref[...]
    o_ref[...] = jax.nn.sigmoid(y)


def kernel(z, batch, W1, b1, W2, b2):
    batch_i = batch.astype(jnp.int32)
    partials = _seg_kernel(z, batch_i)
    tc_sums, tc_cnt = _tc_seg(z, batch_i[:R_TC].reshape(NBLK_TC, 1, B_TC))
    w2p = jnp.pad(W2.T, ((0, 0), (0, 127)))
    b2p = jnp.broadcast_to(b2.reshape(1, 1), (1, 128))
    out = pl.pallas_call(
        _mlp_body,
        out_shape=jax.ShapeDtypeStruct((G, 128), jnp.float32),
    )(partials, tc_sums, tc_cnt, W1, b1.reshape(1, G), w2p, b2p)
    return out[:, :1]


# speedup vs baseline: 1.1058x; 1.1058x over previous
import functools

import jax
import jax.numpy as jnp
from jax import lax
from jax.experimental import pallas as pl
from jax.experimental.pallas import tpu as pltpu
from jax.experimental.pallas import tpu_sc as plsc

N = 100000
D = 512
G = 64
L = 16
NT = 32
ACC_W = D + L

B_TC = 2048
R_TC = 65536
NBLK_TC = R_TC // B_TC
SC_OFF = R_TC
N_SC = N - R_TC

ROWS_PER_CHUNK = 32
N_CHUNKS = N_SC // ROWS_PER_CHUNK
BASE_CHUNKS = N_CHUNKS // NT
EXTRA_CHUNKS = N_CHUNKS % NT
NBUF = 4
MAX_CHUNKS = -(-(BASE_CHUNKS + 1) // NBUF) * NBUF
MAX_ROWS = MAX_CHUNKS * ROWS_PER_CHUNK


def _seg_body(z_hbm, batch_hbm, out_hbm, batch_v, zbuf, acc, *sems):
    cid = lax.axis_index("c")
    sid = lax.axis_index("s")
    wid = sid * 2 + cid
    n_chunks = BASE_CHUNKS + jnp.where(wid < EXTRA_CHUNKS, 1, 0)
    chunk0 = wid * BASE_CHUNKS + jnp.minimum(wid, EXTRA_CHUNKS)
    row0 = SC_OFF + chunk0 * ROWS_PER_CHUNK

    zero16 = jnp.zeros((L,), jnp.float32)

    def _zero_row(i, carry):
        for j in range(ACC_W // L):
            acc[i, pl.ds(j * L, L)] = zero16
        return carry

    lax.fori_loop(0, G + 1, _zero_row, 0)

    bstart = jnp.minimum(row0, N - MAX_ROWS)
    boff = row0 - bstart
    pltpu.sync_copy(batch_hbm.at[pl.ds(bstart, MAX_ROWS)],
                    batch_v.at[pl.ds(0, MAX_ROWS)])

    def _dma(j, buf):
        src = jnp.where(j < n_chunks,
                        SC_OFF + (chunk0 + j) * ROWS_PER_CHUNK,
                        N - ROWS_PER_CHUNK)
        return pltpu.make_async_copy(
            z_hbm.at[pl.ds(src, ROWS_PER_CHUNK)], zbuf.at[buf], sems[buf])

    def _process(i, buf):
        valid = i < n_chunks
        loff = jnp.where(valid, boff + i * ROWS_PER_CHUNK, 0)
        seg_lo = batch_v[pl.ds(loff, L)]
        seg_hi = batch_v[pl.ds(loff + ROWS_PER_CHUNK - L, L)]
        s_first = jnp.where(valid, seg_lo[0], G)
        s_last = jnp.where(valid, seg_hi[L - 1], G)

        @pl.when(s_first == s_last)
        def _fast():
            def _cols(c8, carry):
                for cc in range(8):
                    cs = pl.ds((c8 * 8 + cc) * L, L)
                    vals = [zbuf[buf, r, cs] for r in range(ROWS_PER_CHUNK)]
                    while len(vals) > 1:
                        vals = [vals[k] + vals[k + 1]
                                for k in range(0, len(vals), 2)]
                    acc[s_first, cs] += vals[0]
                return carry

            lax.fori_loop(0, D // L // 8, _cols, 0)
            acc[s_first, pl.ds(D, L)] += jnp.full(
                (L,), float(ROWS_PER_CHUNK), jnp.float32)

        @pl.when(s_first != s_last)
        def _slow():
            def _row(r, carry):
                s_r = batch_v[pl.ds(loff + r, L)][0]

                def _cols(c8, c2):
                    for cc in range(8):
                        cs = pl.ds((c8 * 8 + cc) * L, L)
                        acc[s_r, cs] += zbuf[buf, r, cs]
                    return c2

                lax.fori_loop(0, D // L // 8, _cols, 0)
                acc[s_r, pl.ds(D, L)] += jnp.full((L,), 1.0, jnp.float32)
                return carry

            lax.fori_loop(0, ROWS_PER_CHUNK, _row, 0)

    for b in range(NBUF):
        _dma(b, b).start()

    def _quad(p, carry):
        i0 = NBUF * p
        for b in range(NBUF):
            _dma(i0 + b, b).wait()
            _process(i0 + b, b)

            @pl.when(i0 + b + NBUF < MAX_CHUNKS)
            def _():
                _dma(i0 + b + NBUF, b).start()

        return carry

    lax.fori_loop(0, MAX_CHUNKS // NBUF, _quad, 0)

    pltpu.sync_copy(acc.at[pl.ds(0, G)], out_hbm.at[wid])


_seg_kernel = functools.partial(
    pl.kernel,
    out_type=jax.ShapeDtypeStruct((NT, G, ACC_W), jnp.float32),
    mesh=plsc.VectorSubcoreMesh(core_axis_name="c", subcore_axis_name="s"),
    scratch_types=[
        pltpu.VMEM((MAX_ROWS + L,), jnp.int32),
        pltpu.VMEM((NBUF, ROWS_PER_CHUNK, D), jnp.float32),
        pltpu.VMEM((G + 1, ACC_W), jnp.float32),
    ] + [pltpu.SemaphoreType.DMA] * NBUF,
)(_seg_body)


def _tc_seg_body(b_ref, z_ref, sums_ref, cnt_ref):
    @pl.when(pl.program_id(0) == 0)
    def _():
        sums_ref[...] = jnp.zeros_like(sums_ref)
        cnt_ref[...] = jnp.zeros_like(cnt_ref)

    seg = b_ref[0, 0, :]
    gid = lax.broadcasted_iota(jnp.int32, (G, B_TC), 0)
    onehot = (jnp.broadcast_to(seg[None, :], (G, B_TC)) == gid
              ).astype(jnp.float32)
    sums_ref[...] += lax.dot_general(
        onehot.astype(jnp.bfloat16), z_ref[...].astype(jnp.bfloat16),
        (((1,), (0,)), ((), ())),
        preferred_element_type=jnp.float32)
    cnt128 = onehot[:, 0:128]
    for q in range(1, B_TC // 128):
        cnt128 = cnt128 + onehot[:, q * 128:(q + 1) * 128]
    cnt_ref[...] += cnt128


def _tc_seg(z, batch3d):
    return pl.pallas_call(
        _tc_seg_body,
        grid=(NBLK_TC,),
        in_specs=[
            pl.BlockSpec((1, 1, B_TC), lambda k: (k, 0, 0)),
            pl.BlockSpec((B_TC, D), lambda k: (k, 0)),
        ],
        out_specs=[
            pl.BlockSpec((G, D), lambda k: (0, 0)),
            pl.BlockSpec((G, 128), lambda k: (0, 0)),
        ],
        out_shape=[
            jax.ShapeDtypeStruct((G, D), jnp.float32),
            jax.ShapeDtypeStruct((G, 128), jnp.float32),
        ],
    )(batch3d, z)


def _mlp_body(p_ref, tcs_ref, tcc_ref, w1_ref, b1_ref, w2p_ref, b2p_ref,
              o_ref):
    total = jnp.sum(p_ref[...], axis=0)
    sums = total[:, :D] + tcs_ref[...]
    cnt = total[:, D:D + 1] + jnp.sum(tcc_ref[...], axis=1, keepdims=True)
    mean = sums / jnp.maximum(cnt, 1.0)
    h = lax.dot_general(mean, w1_ref[...], (((1,), (1,)), ((), ())),
                        preferred_element_type=jnp.float32) + b1_ref[...]
    h = jnp.maximum(h, 0.0)
    y = lax.dot_general(h, w2p_ref[...], (((1,), (0,)), ((), ())),
                        preferred_element_type=jnp.float32) + b2p_ref[...]
    o_ref[...] = jax.nn.sigmoid(y)


def kernel(z, batch, W1, b1, W2, b2):
    batch_i = batch.astype(jnp.int32)
    partials = _seg_kernel(z, batch_i)
    tc_sums, tc_cnt = _tc_seg(z, batch_i[:R_TC].reshape(NBLK_TC, 1, B_TC))
    w2p = jnp.pad(W2.T, ((0, 0), (0, 127)))
    b2p = jnp.broadcast_to(b2.reshape(1, 1), (1, 128))
    out = pl.pallas_call(
        _mlp_body,
        out_shape=jax.ShapeDtypeStruct((G, 128), jnp.float32),
    )(partials, tc_sums, tc_cnt, W1, b1.reshape(1, G), w2p, b2p)
    return out[:, :1]


# speedup vs baseline: 1.1331x; 1.0247x over previous
import functools

import jax
import jax.numpy as jnp
from jax import lax
from jax.experimental import pallas as pl
from jax.experimental.pallas import tpu as pltpu
from jax.experimental.pallas import tpu_sc as plsc

N = 100000
D = 512
G = 64
L = 16
NT = 32
ACC_W = D + L

B_TC = 2048
R_TC = 63488
NBLK_TC = R_TC // B_TC
SC_OFF = R_TC
N_SC = N - R_TC

ROWS_PER_CHUNK = 32
N_CHUNKS = N_SC // ROWS_PER_CHUNK
BASE_CHUNKS = N_CHUNKS // NT
EXTRA_CHUNKS = N_CHUNKS % NT
NBUF = 4
MAX_CHUNKS = -(-(BASE_CHUNKS + 1) // NBUF) * NBUF
MAX_ROWS = MAX_CHUNKS * ROWS_PER_CHUNK


def _seg_body(z_hbm, batch_hbm, out_hbm, batch_v, zbuf, acc, *sems):
    cid = lax.axis_index("c")
    sid = lax.axis_index("s")
    wid = sid * 2 + cid
    n_chunks = BASE_CHUNKS + jnp.where(wid < EXTRA_CHUNKS, 1, 0)
    chunk0 = wid * BASE_CHUNKS + jnp.minimum(wid, EXTRA_CHUNKS)
    row0 = SC_OFF + chunk0 * ROWS_PER_CHUNK

    zero16 = jnp.zeros((L,), jnp.float32)

    def _zero_row(i, carry):
        for j in range(ACC_W // L):
            acc[i, pl.ds(j * L, L)] = zero16
        return carry

    lax.fori_loop(0, G + 1, _zero_row, 0)

    bstart = jnp.minimum(row0, N - MAX_ROWS)
    boff = row0 - bstart
    pltpu.sync_copy(batch_hbm.at[pl.ds(bstart, MAX_ROWS)],
                    batch_v.at[pl.ds(0, MAX_ROWS)])

    def _dma(j, buf):
        src = jnp.where(j < n_chunks,
                        SC_OFF + (chunk0 + j) * ROWS_PER_CHUNK,
                        N - ROWS_PER_CHUNK)
        return pltpu.make_async_copy(
            z_hbm.at[pl.ds(src, ROWS_PER_CHUNK)], zbuf.at[buf], sems[buf])

    def _process(i, buf):
        valid = i < n_chunks
        loff = jnp.where(valid, boff + i * ROWS_PER_CHUNK, 0)
        seg_lo = batch_v[pl.ds(loff, L)]
        seg_hi = batch_v[pl.ds(loff + ROWS_PER_CHUNK - L, L)]
        s_first = jnp.where(valid, seg_lo[0], G)
        s_last = jnp.where(valid, seg_hi[L - 1], G)

        @pl.when(s_first == s_last)
        def _fast():
            def _cols(c8, carry):
                for cc in range(8):
                    cs = pl.ds((c8 * 8 + cc) * L, L)
                    vals = [zbuf[buf, r, cs] for r in range(ROWS_PER_CHUNK)]
                    while len(vals) > 1:
                        vals = [vals[k] + vals[k + 1]
                                for k in range(0, len(vals), 2)]
                    acc[s_first, cs] += vals[0]
                return carry

            lax.fori_loop(0, D // L // 8, _cols, 0)
            acc[s_first, pl.ds(D, L)] += jnp.full(
                (L,), float(ROWS_PER_CHUNK), jnp.float32)

        @pl.when(s_first != s_last)
        def _slow():
            def _row(r, carry):
                s_r = batch_v[pl.ds(loff + r, L)][0]

                def _cols(c8, c2):
                    for cc in range(8):
                        cs = pl.ds((c8 * 8 + cc) * L, L)
                        acc[s_r, cs] += zbuf[buf, r, cs]
                    return c2

                lax.fori_loop(0, D // L // 8, _cols, 0)
                acc[s_r, pl.ds(D, L)] += jnp.full((L,), 1.0, jnp.float32)
                return carry

            lax.fori_loop(0, ROWS_PER_CHUNK, _row, 0)

    for b in range(NBUF):
        _dma(b, b).start()

    def _quad(p, carry):
        i0 = NBUF * p
        for b in range(NBUF):
            _dma(i0 + b, b).wait()
            _process(i0 + b, b)

            @pl.when(i0 + b + NBUF < MAX_CHUNKS)
            def _():
                _dma(i0 + b + NBUF, b).start()

        return carry

    lax.fori_loop(0, MAX_CHUNKS // NBUF, _quad, 0)

    pltpu.sync_copy(acc.at[pl.ds(0, G)], out_hbm.at[wid])


_seg_kernel = functools.partial(
    pl.kernel,
    out_type=jax.ShapeDtypeStruct((NT, G, ACC_W), jnp.float32),
    mesh=plsc.VectorSubcoreMesh(core_axis_name="c", subcore_axis_name="s"),
    scratch_types=[
        pltpu.VMEM((MAX_ROWS + L,), jnp.int32),
        pltpu.VMEM((NBUF, ROWS_PER_CHUNK, D), jnp.float32),
        pltpu.VMEM((G + 1, ACC_W), jnp.float32),
    ] + [pltpu.SemaphoreType.DMA] * NBUF,
)(_seg_body)


def _tc_seg_body(b_ref, z_ref, sums_ref, cnt_ref):
    @pl.when(pl.program_id(0) == 0)
    def _():
        sums_ref[...] = jnp.zeros_like(sums_ref)
        cnt_ref[...] = jnp.zeros_like(cnt_ref)

    seg = b_ref[0, 0, :]
    gid = lax.broadcasted_iota(jnp.int32, (G, B_TC), 0)
    onehot = (jnp.broadcast_to(seg[None, :], (G, B_TC)) == gid
              ).astype(jnp.float32)
    sums_ref[...] += lax.dot_general(
        onehot.astype(jnp.bfloat16), z_ref[...].astype(jnp.bfloat16),
        (((1,), (0,)), ((), ())),
        preferred_element_type=jnp.float32)
    cnt128 = onehot[:, 0:128]
    for q in range(1, B_TC // 128):
        cnt128 = cnt128 + onehot[:, q * 128:(q + 1) * 128]
    cnt_ref[...] += cnt128


def _tc_seg(z, batch3d):
    return pl.pallas_call(
        _tc_seg_body,
        grid=(NBLK_TC,),
        in_specs=[
            pl.BlockSpec((1, 1, B_TC), lambda k: (k, 0, 0)),
            pl.BlockSpec((B_TC, D), lambda k: (k, 0)),
        ],
        out_specs=[
            pl.BlockSpec((G, D), lambda k: (0, 0)),
            pl.BlockSpec((G, 128), lambda k: (0, 0)),
        ],
        out_shape=[
            jax.ShapeDtypeStruct((G, D), jnp.float32),
            jax.ShapeDtypeStruct((G, 128), jnp.float32),
        ],
    )(batch3d, z)


def _mlp_body(p_ref, tcs_ref, tcc_ref, w1_ref, b1_ref, w2p_ref, b2p_ref,
              o_ref):
    total = jnp.sum(p_ref[...], axis=0)
    sums = total[:, :D] + tcs_ref[...]
    cnt = total[:, D:D + 1] + jnp.sum(tcc_ref[...], axis=1, keepdims=True)
    mean = sums / jnp.maximum(cnt, 1.0)
    h = lax.dot_general(mean, w1_ref[...], (((1,), (1,)), ((), ())),
                        preferred_element_type=jnp.float32) + b1_ref[...]
    h = jnp.maximum(h, 0.0)
    y = lax.dot_general(h, w2p_ref[...], (((1,), (0,)), ((), ())),
                        preferred_element_type=jnp.float32) + b2p_ref[...]
    o_ref[...] = jax.nn.sigmoid(y)


def kernel(z, batch, W1, b1, W2, b2):
    batch_i = batch.astype(jnp.int32)
    partials = _seg_kernel(z, batch_i)
    tc_sums, tc_cnt = _tc_seg(z, batch_i[:R_TC].reshape(NBLK_TC, 1, B_TC))
    w2p = jnp.pad(W2.T, ((0, 0), (0, 127)))
    b2p = jnp.broadcast_to(b2.reshape(1, 1), (1, 128))
    out = pl.pallas_call(
        _mlp_body,
        out_shape=jax.ShapeDtypeStruct((G, 128), jnp.float32),
    )(partials, tc_sums, tc_cnt, W1, b1.reshape(1, G), w2p, b2p)
    return out[:, :1]
